# baseline (speedup 1.0000x reference)
import functools

import jax
import jax.numpy as jnp
from jax import lax
from jax.experimental import pallas as pl
from jax.experimental.pallas import tpu as pltpu
from jax.experimental.pallas import tpu_sc as plsc


@functools.lru_cache(maxsize=None)
def _build_sc_gather(rows, D):
    info = plsc.get_sparse_core_info()
    NW = info.num_cores * info.num_subcores
    assert rows % (8 * NW) == 0
    b_per_w = rows // NW

    ch = b_per_w
    while ch * D * 4 > 220_000 or ch % 8 != 0:
        ch //= 2
    n_ch = b_per_w // ch

    mesh = plsc.VectorSubcoreMesh(core_axis_name="c", subcore_axis_name="s")

    @functools.partial(
        pl.kernel,
        out_type=jax.ShapeDtypeStruct((rows, D), jnp.float32),
        mesh=mesh,
        scratch_types=[
            pltpu.VMEM((b_per_w,), jnp.int32),
            pltpu.VMEM((ch, D), jnp.float32),
            pltpu.VMEM((ch, D), jnp.float32),
            pltpu.SemaphoreType.DMA,
            pltpu.SemaphoreType.DMA,
        ],
        compiler_params=pltpu.CompilerParams(use_tc_tiling_on_sc=False),
    )
    def sc_gather(table_hbm, idx_hbm, out_hbm, idx_v, buf0, buf1, sem0, sem1):
        wid = lax.axis_index("s") * info.num_cores + lax.axis_index("c")
        base = wid * b_per_w
        pltpu.sync_copy(idx_hbm.at[pl.ds(base, b_per_w)], idx_v)

        bufs = (buf0, buf1)
        sems = (sem0, sem1)

        def gather(c):
            return pltpu.async_copy(
                table_hbm.at[idx_v.at[pl.ds(c * ch, ch)]],
                bufs[c % 2],
                sems[c % 2],
            )

        cur = gather(0)
        for c in range(n_ch):
            nxt = gather(c + 1) if c + 1 < n_ch else None
            cur.wait()
            pltpu.sync_copy(bufs[c % 2], out_hbm.at[pl.ds(base + c * ch, ch)])
            cur = nxt

    return sc_gather


def kernel(x, indices):
    B, N, D = x.shape
    K = indices.shape[0]
    table = x.reshape(B * N, D)
    flat_idx = (
        jnp.arange(B, dtype=jnp.int32)[:, None] * N
        + indices[None, :].astype(jnp.int32)
    ).reshape(-1)
    rows = B * K
    out = _build_sc_gather(rows, D)(table, flat_idx)
    return out.reshape(B, K, D)


# speedup vs baseline: 1.3390x; 1.3390x over previous
import functools

import jax
import jax.numpy as jnp
from jax import lax
from jax.experimental import pallas as pl
from jax.experimental.pallas import tpu as pltpu
from jax.experimental.pallas import tpu_sc as plsc

_LANES = 16


@functools.lru_cache(maxsize=None)
def _build_sc_gather(B, N, D, K):
    info = plsc.get_sparse_core_info()
    NW = info.num_cores * info.num_subcores
    assert B % NW == 0
    nb = B // NW
    kpad = -(-K // _LANES) * _LANES

    mesh = plsc.VectorSubcoreMesh(core_axis_name="c", subcore_axis_name="s")

    @functools.partial(
        pl.kernel,
        out_type=jax.ShapeDtypeStruct((B, K, D), jnp.float32),
        mesh=mesh,
        scratch_types=[
            pltpu.VMEM((kpad,), jnp.int32),
            pltpu.VMEM((nb, D), jnp.float32),
            pltpu.VMEM((nb, D), jnp.float32),
            pltpu.SemaphoreType.DMA,
            pltpu.SemaphoreType.DMA,
        ],
        compiler_params=pltpu.CompilerParams(
            use_tc_tiling_on_sc=True, needs_layout_passes=False
        ),
    )
    def sc_gather(x_hbm, idx_hbm, out_hbm, idx_v, buf0, buf1, sem0, sem1):
        wid = lax.axis_index("s") * info.num_cores + lax.axis_index("c")
        b0 = wid * nb
        pltpu.sync_copy(idx_hbm, idx_v)

        lane_ids = lax.iota(jnp.int32, _LANES)
        vecs = [idx_v[pl.ds(g * _LANES, _LANES)] for g in range(kpad // _LANES)]
        idx_scalars = [
            jnp.sum(jnp.where(lane_ids == (j % _LANES), vecs[j // _LANES], 0))
            for j in range(K)
        ]

        bufs = (buf0, buf1)
        sems = (sem0, sem1)

        def fire(j):
            return pltpu.async_copy(
                x_hbm.at[pl.ds(b0, nb), idx_scalars[j]], bufs[j % 2], sems[j % 2]
            )

        cur = fire(0)
        for j in range(K):
            nxt = fire(j + 1) if j + 1 < K else None
            cur.wait()
            pltpu.sync_copy(bufs[j % 2], out_hbm.at[pl.ds(b0, nb), j])
            cur = nxt

    return sc_gather


def kernel(x, indices):
    B, N, D = x.shape
    K = indices.shape[0]
    kpad = -(-K // _LANES) * _LANES
    idx_pad = jnp.zeros((kpad,), jnp.int32).at[:K].set(indices.astype(jnp.int32))
    return _build_sc_gather(B, N, D, K)(x, idx_pad)


# speedup vs baseline: 7.5111x; 5.6093x over previous
import functools

import jax
import jax.numpy as jnp
from jax import lax
from jax.experimental import pallas as pl
from jax.experimental.pallas import tpu as pltpu
from jax.experimental.pallas import tpu_sc as plsc

_LANES = 16


@functools.lru_cache(maxsize=None)
def _build_sc_gather(N, D, B, K):
    info = plsc.get_sparse_core_info()
    NW = info.num_cores * info.num_subcores
    assert B % (NW * 128) == 0
    nc = B // NW
    kpad = -(-K // _LANES) * _LANES

    mesh = plsc.VectorSubcoreMesh(core_axis_name="c", subcore_axis_name="s")

    @functools.partial(
        pl.kernel,
        out_type=jax.ShapeDtypeStruct((K, D, B), jnp.float32),
        mesh=mesh,
        scratch_types=[
            pltpu.VMEM((kpad,), jnp.int32),
            pltpu.VMEM((D, nc), jnp.float32),
            pltpu.VMEM((D, nc), jnp.float32),
            pltpu.SemaphoreType.DMA,
            pltpu.SemaphoreType.DMA,
        ],
        compiler_params=pltpu.CompilerParams(
            use_tc_tiling_on_sc=True, needs_layout_passes=False
        ),
    )
    def sc_gather(x_hbm, idx_hbm, out_hbm, idx_v, buf0, buf1, sem0, sem1):
        wid = lax.axis_index("s") * info.num_cores + lax.axis_index("c")
        c0 = wid * nc
        pltpu.sync_copy(idx_hbm, idx_v)

        lane_ids = lax.iota(jnp.int32, _LANES)
        vecs = [idx_v[pl.ds(g * _LANES, _LANES)] for g in range(kpad // _LANES)]
        idx_scalars = [
            jnp.sum(jnp.where(lane_ids == (j % _LANES), vecs[j // _LANES], 0))
            for j in range(K)
        ]

        bufs = (buf0, buf1)
        sems = (sem0, sem1)

        def fire(j):
            return pltpu.async_copy(
                x_hbm.at[idx_scalars[j], :, pl.ds(c0, nc)],
                bufs[j % 2],
                sems[j % 2],
            )

        cur = fire(0)
        for j in range(K):
            nxt = fire(j + 1) if j + 1 < K else None
            cur.wait()
            pltpu.sync_copy(bufs[j % 2], out_hbm.at[j, :, pl.ds(c0, nc)])
            cur = nxt

    return sc_gather


def kernel(x, indices):
    B, N, D = x.shape
    K = indices.shape[0]
    kpad = -(-K // _LANES) * _LANES
    idx_pad = jnp.zeros((kpad,), jnp.int32).at[:K].set(indices.astype(jnp.int32))
    x3 = x.transpose(1, 2, 0)
    out3 = _build_sc_gather(N, D, B, K)(x3, idx_pad)
    return out3.transpose(2, 0, 1)
